# Initial kernel scaffold; baseline (speedup 1.0000x reference)
import jax
import jax.numpy as jnp
from jax.experimental import pallas as pl


def kernel(x, edge_index, batch, bn_feat_g, bn_feat_b, W_feat, bnc_g0, bnc_b0, Wc0, bc0, bnc_g1, bnc_b1, Wc1, bc1, bnc_g2, bnc_b2, Wc2, bc2, bn_fc_g, bn_fc_b, W_fc, b_fc, bn_hid_g, bn_hid_b, W_cls, b_cls):
    raise NotImplementedError("write your pallas kernel here")


# speedup vs baseline: 16.1683x; 16.1683x over previous
import functools

import jax
import jax.numpy as jnp
from jax import lax
from jax.experimental import pallas as pl
from jax.experimental.pallas import tpu as pltpu
from jax.experimental.pallas import tpu_sc as plsc

N = 10000
E = 320000
F = 128
NG = 64
NCLS = 10
EPS = 1e-5

NC, NS = 2, 16
NW = NC * NS
EW = E // NW
FH = F // 2
ECS = E // NS
CH = 800
NCHUNK = ECS // CH
APAD = 10240
RPS = APAD // NS
DEG_R, DEG_C = 16, 1024

BLK = 1000
GRID = N // BLK

_mesh = plsc.VectorSubcoreMesh(core_axis_name="core", subcore_axis_name="sub",
                               num_cores=NC, num_subcores=NS)


NPAD = DEG_R * DEG_C
DSLICE = NPAD // NS


@functools.partial(
    pl.kernel,
    out_type=jax.ShapeDtypeStruct((NC, NPAD), jnp.float32),
    mesh=_mesh,
    scratch_types=[
        pltpu.VMEM((EW,), jnp.int32),
        pltpu.VMEM((NPAD,), jnp.float32),
        pltpu.VMEM((NS, DSLICE), jnp.float32),
        pltpu.VMEM((DSLICE,), jnp.float32),
        pltpu.VMEM_SHARED((NS, NPAD), jnp.float32),
    ],
    compiler_params=pltpu.CompilerParams(needs_layout_passes=False),
)
def _deg_call(r_hbm, out_hbm, ridx_v, deg_v, part_v, res_v, deg_sp):
    cid = lax.axis_index("core")
    sid = lax.axis_index("sub")

    def _z(k, _):
        deg_v[pl.ds(k * 16, 16)] = jnp.zeros((16,), jnp.float32)
        return 0
    lax.fori_loop(0, NPAD // 16, _z, 0)

    pltpu.sync_copy(r_hbm.at[pl.ds((cid * NS + sid) * EW, EW)], ridx_v)
    ones = jnp.ones((16,), jnp.float32)

    def _hist(k, _):
        idx = ridx_v[pl.ds(k * 16, 16)]
        plsc.addupdate_scatter(deg_v, [idx], ones)
        return 0
    lax.fori_loop(0, EW // 16, _hist, 0)

    pltpu.sync_copy(deg_v, deg_sp.at[sid])
    plsc.subcore_barrier()

    for p in range(NS):
        pltpu.sync_copy(deg_sp.at[p, pl.ds(sid * DSLICE, DSLICE)], part_v.at[p])

    def _red(j, _):
        acc = jnp.zeros((16,), jnp.float32)
        for p in range(NS):
            acc = acc + part_v[p, pl.ds(j * 16, 16)]
        res_v[pl.ds(j * 16, 16)] = acc
        return 0
    lax.fori_loop(0, DSLICE // 16, _red, 0)
    pltpu.sync_copy(res_v, out_hbm.at[cid, pl.ds(sid * DSLICE, DSLICE)])


@functools.partial(
    pl.kernel,
    out_type=jax.ShapeDtypeStruct((NC, APAD, FH), jnp.float32),
    mesh=_mesh,
    scratch_types=[
        pltpu.VMEM((CH,), jnp.int32),
        pltpu.VMEM((CH,), jnp.int32),
        pltpu.VMEM((CH, FH), jnp.float32),
        pltpu.VMEM_SHARED((APAD, FH), jnp.float32),
        pltpu.SemaphoreType.DMA,
    ],
    compiler_params=pltpu.CompilerParams(use_tc_tiling_on_sc=False),
)
def _agg_call(hp_hbm, r_hbm, c_hbm, out_hbm, ridx_v, cidx_v, rows_v, agg_sp, sem):
    cid = lax.axis_index("core")
    sid = lax.axis_index("sub")
    hp_my = hp_hbm.at[cid]

    def _zrow(i, _):
        for j in range(FH // 16):
            rows_v[i, pl.ds(j * 16, 16)] = jnp.zeros((16,), jnp.float32)
        return 0
    lax.fori_loop(0, RPS, _zrow, 0)
    pltpu.sync_copy(rows_v.at[pl.ds(0, RPS)], agg_sp.at[pl.ds(sid * RPS, RPS)])
    plsc.subcore_barrier()

    base0 = sid * ECS

    def _chunk(g, _):
        base = base0 + g * CH
        pltpu.sync_copy(r_hbm.at[pl.ds(base, CH)], ridx_v)
        pltpu.sync_copy(c_hbm.at[pl.ds(base, CH)], cidx_v)
        pltpu.async_copy(hp_my.at[ridx_v], rows_v, sem).wait()
        pltpu.sync_copy(rows_v, agg_sp.at[cidx_v], add=True)
        return 0
    lax.fori_loop(0, NCHUNK, _chunk, 0)
    plsc.subcore_barrier()

    row0 = sid * RPS
    pltpu.sync_copy(agg_sp.at[pl.ds(row0, RPS)], rows_v.at[pl.ds(0, RPS)])
    pltpu.sync_copy(rows_v.at[pl.ds(0, RPS)], out_hbm.at[cid, pl.ds(row0, RPS)])


def _stats_body(x_ref, s1_ref, s2_ref):
    i = pl.program_id(0)

    @pl.when(i == 0)
    def _():
        s1_ref[...] = jnp.zeros_like(s1_ref)
        s2_ref[...] = jnp.zeros_like(s2_ref)
    xb = x_ref[...]
    s1_ref[...] += jnp.sum(xb, axis=0, keepdims=True)
    s2_ref[...] += jnp.sum(xb * xb, axis=0, keepdims=True)


_stats_call = pl.pallas_call(
    _stats_body,
    grid=(GRID,),
    in_specs=[pl.BlockSpec((BLK, F), lambda i: (i, 0))],
    out_specs=[pl.BlockSpec((1, F), lambda i: (0, 0)),
               pl.BlockSpec((1, F), lambda i: (0, 0))],
    out_shape=[jax.ShapeDtypeStruct((1, F), jnp.float32),
               jax.ShapeDtypeStruct((1, F), jnp.float32)],
)


def _bn_coeffs(s1, s2, g, b):
    m = s1 * (1.0 / N)
    v = s2 * (1.0 / N) - m * m
    sc = g * lax.rsqrt(v + EPS)
    return sc, b - m * sc


def _feat_body(x_ref, s1_ref, s2_ref, g_ref, b_ref, w_ref, h_ref, t1_ref, t2_ref):
    i = pl.program_id(0)
    sc, sh = _bn_coeffs(s1_ref[...], s2_ref[...], g_ref[...], b_ref[...])
    t = x_ref[...] * sc + sh
    h = jnp.maximum(jnp.dot(t, w_ref[...], preferred_element_type=jnp.float32), 0.0)
    h_ref[...] = h

    @pl.when(i == 0)
    def _():
        t1_ref[...] = jnp.zeros_like(t1_ref)
        t2_ref[...] = jnp.zeros_like(t2_ref)
    t1_ref[...] += jnp.sum(h, axis=0, keepdims=True)
    t2_ref[...] += jnp.sum(h * h, axis=0, keepdims=True)


_feat_call = pl.pallas_call(
    _feat_body,
    grid=(GRID,),
    in_specs=[pl.BlockSpec((BLK, F), lambda i: (i, 0)),
              pl.BlockSpec((1, F), lambda i: (0, 0)),
              pl.BlockSpec((1, F), lambda i: (0, 0)),
              pl.BlockSpec((1, F), lambda i: (0, 0)),
              pl.BlockSpec((1, F), lambda i: (0, 0)),
              pl.BlockSpec((F, F), lambda i: (0, 0))],
    out_specs=[pl.BlockSpec((BLK, F), lambda i: (i, 0)),
               pl.BlockSpec((1, F), lambda i: (0, 0)),
               pl.BlockSpec((1, F), lambda i: (0, 0))],
    out_shape=[jax.ShapeDtypeStruct((N, F), jnp.float32),
               jax.ShapeDtypeStruct((1, F), jnp.float32),
               jax.ShapeDtypeStruct((1, F), jnp.float32)],
)


def _pre_body(h_ref, s1_ref, s2_ref, g_ref, b_ref, w_ref, d0_ref, d1_ref, hp_ref):
    sc, sh = _bn_coeffs(s1_ref[...], s2_ref[...], g_ref[...], b_ref[...])
    t = h_ref[...] * sc + sh
    u = jnp.dot(t, w_ref[...], preferred_element_type=jnp.float32)
    dis = lax.rsqrt(d0_ref[...] + d1_ref[...] + 1.0)
    hp = u * dis
    hp_ref[0] = hp[:, :FH]
    hp_ref[1] = hp[:, FH:]


_pre_call = pl.pallas_call(
    _pre_body,
    grid=(GRID,),
    in_specs=[pl.BlockSpec((BLK, F), lambda i: (i, 0)),
              pl.BlockSpec((1, F), lambda i: (0, 0)),
              pl.BlockSpec((1, F), lambda i: (0, 0)),
              pl.BlockSpec((1, F), lambda i: (0, 0)),
              pl.BlockSpec((1, F), lambda i: (0, 0)),
              pl.BlockSpec((F, F), lambda i: (0, 0)),
              pl.BlockSpec((BLK, 1), lambda i: (i, 0)),
              pl.BlockSpec((BLK, 1), lambda i: (i, 0))],
    out_specs=pl.BlockSpec((2, BLK, FH), lambda i: (0, i, 0)),
    out_shape=jax.ShapeDtypeStruct((2, N, FH), jnp.float32),
)


def _post_body(a_ref, hp_ref, d0_ref, d1_ref, b_ref, h_ref, t1_ref, t2_ref):
    i = pl.program_id(0)
    dis = lax.rsqrt(d0_ref[...] + d1_ref[...] + 1.0)
    agg = jnp.concatenate([a_ref[0], a_ref[1]], axis=1)
    hp = jnp.concatenate([hp_ref[0], hp_ref[1]], axis=1)
    h = jnp.maximum((agg + hp) * dis + b_ref[...], 0.0)
    h_ref[...] = h

    @pl.when(i == 0)
    def _():
        t1_ref[...] = jnp.zeros_like(t1_ref)
        t2_ref[...] = jnp.zeros_like(t2_ref)
    t1_ref[...] += jnp.sum(h, axis=0, keepdims=True)
    t2_ref[...] += jnp.sum(h * h, axis=0, keepdims=True)


_post_call = pl.pallas_call(
    _post_body,
    grid=(GRID,),
    in_specs=[pl.BlockSpec((2, BLK, FH), lambda i: (0, i, 0)),
              pl.BlockSpec((2, BLK, FH), lambda i: (0, i, 0)),
              pl.BlockSpec((BLK, 1), lambda i: (i, 0)),
              pl.BlockSpec((BLK, 1), lambda i: (i, 0)),
              pl.BlockSpec((1, F), lambda i: (0, 0))],
    out_specs=[pl.BlockSpec((BLK, F), lambda i: (i, 0)),
               pl.BlockSpec((1, F), lambda i: (0, 0)),
               pl.BlockSpec((1, F), lambda i: (0, 0))],
    out_shape=[jax.ShapeDtypeStruct((N, F), jnp.float32),
               jax.ShapeDtypeStruct((1, F), jnp.float32),
               jax.ShapeDtypeStruct((1, F), jnp.float32)],
)


def _pool_body(h_ref, bat_ref, p_ref):
    i = pl.program_id(0)
    bat = bat_ref[0]
    gid = lax.broadcasted_iota(jnp.int32, (NG, BLK), 0)
    mask = jnp.where(bat == gid, 1.0, 0.0)

    @pl.when(i == 0)
    def _():
        p_ref[...] = jnp.zeros_like(p_ref)
    p_ref[...] += jnp.dot(mask, h_ref[...], preferred_element_type=jnp.float32)


_pool_call = pl.pallas_call(
    _pool_body,
    grid=(GRID,),
    in_specs=[pl.BlockSpec((BLK, F), lambda i: (i, 0)),
              pl.BlockSpec((1, 1, BLK), lambda i: (i, 0, 0))],
    out_specs=pl.BlockSpec((NG, F), lambda i: (0, 0)),
    out_shape=jax.ShapeDtypeStruct((NG, F), jnp.float32),
)


def _head_body(p_ref, g1_ref, b1_ref, wf_ref, bf_ref, g2_ref, b2_ref,
               wc_ref, bc_ref, o_ref):
    p = p_ref[...]
    m = jnp.mean(p, axis=0, keepdims=True)
    v = jnp.mean((p - m) * (p - m), axis=0, keepdims=True)
    h = g1_ref[...] * (p - m) * lax.rsqrt(v + EPS) + b1_ref[...]
    h = jnp.maximum(jnp.dot(h, wf_ref[...], preferred_element_type=jnp.float32)
                    + bf_ref[...], 0.0)
    m2 = jnp.mean(h, axis=0, keepdims=True)
    v2 = jnp.mean((h - m2) * (h - m2), axis=0, keepdims=True)
    h = g2_ref[...] * (h - m2) * lax.rsqrt(v2 + EPS) + b2_ref[...]
    lg = jnp.dot(h, wc_ref[...], preferred_element_type=jnp.float32) + bc_ref[...]
    mx = jnp.max(lg, axis=-1, keepdims=True)
    lse = jnp.log(jnp.sum(jnp.exp(lg - mx), axis=-1, keepdims=True)) + mx
    o_ref[...] = lg - lse


_head_call = pl.pallas_call(
    _head_body,
    out_shape=jax.ShapeDtypeStruct((NG, NCLS), jnp.float32),
)


def kernel(x, edge_index, batch, bn_feat_g, bn_feat_b, W_feat,
           bnc_g0, bnc_b0, Wc0, bc0, bnc_g1, bnc_b1, Wc1, bc1,
           bnc_g2, bnc_b2, Wc2, bc2, bn_fc_g, bn_fc_b, W_fc, b_fc,
           bn_hid_g, bn_hid_b, W_cls, b_cls):
    r = edge_index[0]
    c = edge_index[1]

    degp = _deg_call(r)
    d0 = degp[0].reshape(NPAD, 1)[:N]
    d1 = degp[1].reshape(NPAD, 1)[:N]

    s1, s2 = _stats_call(x)
    h, t1, t2 = _feat_call(x, s1, s2, bn_feat_g.reshape(1, F),
                           bn_feat_b.reshape(1, F), W_feat)

    gstk = jnp.stack([bnc_g0, bnc_g1, bnc_g2]).reshape(3, 1, F)
    bstk = jnp.stack([bnc_b0, bnc_b1, bnc_b2]).reshape(3, 1, F)
    Wstk = jnp.stack([Wc0, Wc1, Wc2])
    bbstk = jnp.stack([bc0, bc1, bc2]).reshape(3, 1, F)

    def _layer(carry, xs):
        hc, t1c, t2c = carry
        g, b, W, bb = xs
        hp = _pre_call(hc, t1c, t2c, g, b, W, d0, d1)
        aggp = _agg_call(hp, r, c)
        return tuple(_post_call(aggp[:, :N], hp, d0, d1, bb)), None

    (h, t1, t2), _ = lax.scan(_layer, (h, t1, t2), (gstk, bstk, Wstk, bbstk))

    bat3 = batch.reshape(GRID, 1, BLK)
    pooled = _pool_call(h, bat3)

    return _head_call(pooled, bn_fc_g.reshape(1, F), bn_fc_b.reshape(1, F),
                      W_fc, b_fc.reshape(1, F), bn_hid_g.reshape(1, F),
                      bn_hid_b.reshape(1, F), W_cls, b_cls.reshape(1, NCLS))


# speedup vs baseline: 17.8863x; 1.1063x over previous
import functools

import jax
import jax.numpy as jnp
from jax import lax
from jax.experimental import pallas as pl
from jax.experimental.pallas import tpu as pltpu
from jax.experimental.pallas import tpu_sc as plsc

N = 10000
E = 320000
F = 128
NG = 64
NCLS = 10
EPS = 1e-5

NC, NS = 2, 16
NW = NC * NS
EW = E // NW
FH = F // 2
ECS = E // NS
CH = 400
NPAIR = ECS // (2 * CH)
APAD = 10240
RPS = APAD // NS
DEG_R, DEG_C = 16, 1024

BLK = 1000
GRID = N // BLK

_mesh = plsc.VectorSubcoreMesh(core_axis_name="core", subcore_axis_name="sub",
                               num_cores=NC, num_subcores=NS)


NPAD = DEG_R * DEG_C
DSLICE = NPAD // NS


@functools.partial(
    pl.kernel,
    out_type=jax.ShapeDtypeStruct((NC, NPAD), jnp.float32),
    mesh=_mesh,
    scratch_types=[
        pltpu.VMEM((EW,), jnp.int32),
        pltpu.VMEM((NPAD,), jnp.float32),
        pltpu.VMEM((NS, DSLICE), jnp.float32),
        pltpu.VMEM((DSLICE,), jnp.float32),
        pltpu.VMEM_SHARED((NS, NPAD), jnp.float32),
    ],
    compiler_params=pltpu.CompilerParams(needs_layout_passes=False),
)
def _deg_call(r_hbm, out_hbm, ridx_v, deg_v, part_v, res_v, deg_sp):
    cid = lax.axis_index("core")
    sid = lax.axis_index("sub")

    def _z(k, _):
        deg_v[pl.ds(k * 16, 16)] = jnp.zeros((16,), jnp.float32)
        return 0
    lax.fori_loop(0, NPAD // 16, _z, 0)

    pltpu.sync_copy(r_hbm.at[pl.ds((cid * NS + sid) * EW, EW)], ridx_v)
    ones = jnp.ones((16,), jnp.float32)

    def _hist(k, _):
        idx = ridx_v[pl.ds(k * 16, 16)]
        plsc.addupdate_scatter(deg_v, [idx], ones)
        return 0
    lax.fori_loop(0, EW // 16, _hist, 0)

    pltpu.sync_copy(deg_v, deg_sp.at[sid])
    plsc.subcore_barrier()

    for p in range(NS):
        pltpu.sync_copy(deg_sp.at[p, pl.ds(sid * DSLICE, DSLICE)], part_v.at[p])

    def _red(j, _):
        acc = jnp.zeros((16,), jnp.float32)
        for p in range(NS):
            acc = acc + part_v[p, pl.ds(j * 16, 16)]
        res_v[pl.ds(j * 16, 16)] = acc
        return 0
    lax.fori_loop(0, DSLICE // 16, _red, 0)
    pltpu.sync_copy(res_v, out_hbm.at[cid, pl.ds(sid * DSLICE, DSLICE)])


@functools.partial(
    pl.kernel,
    out_type=jax.ShapeDtypeStruct((NC, APAD, FH), jnp.float32),
    mesh=_mesh,
    scratch_types=[
        pltpu.VMEM((CH,), jnp.int32),
        pltpu.VMEM((CH,), jnp.int32),
        pltpu.VMEM((CH,), jnp.int32),
        pltpu.VMEM((CH,), jnp.int32),
        pltpu.VMEM((CH, FH), jnp.float32),
        pltpu.VMEM((CH, FH), jnp.float32),
        pltpu.SemaphoreType.DMA,
        pltpu.SemaphoreType.DMA,
        pltpu.SemaphoreType.DMA,
        pltpu.SemaphoreType.DMA,
        pltpu.VMEM_SHARED((APAD, FH), jnp.float32),
    ],
    compiler_params=pltpu.CompilerParams(use_tc_tiling_on_sc=False),
)
def _agg_call(hp_hbm, r_hbm, c_hbm, out_hbm, ridx0_v, ridx1_v, cidx0_v, cidx1_v,
              rows0_v, rows1_v, sg0, sg1, ss0, ss1, agg_sp):
    cid = lax.axis_index("core")
    sid = lax.axis_index("sub")
    hp_my = hp_hbm.at[cid]
    ridx = (ridx0_v, ridx1_v)
    cidx = (cidx0_v, cidx1_v)
    rows = (rows0_v, rows1_v)
    sg = (sg0, sg1)
    ss = (ss0, ss1)

    def _zrow(i, _):
        for j in range(FH // 16):
            rows0_v[i, pl.ds(j * 16, 16)] = jnp.zeros((16,), jnp.float32)
        return 0
    lax.fori_loop(0, RPS // 2, _zrow, 0)
    pltpu.sync_copy(rows0_v.at[pl.ds(0, RPS // 2)],
                    agg_sp.at[pl.ds(sid * RPS, RPS // 2)])
    pltpu.sync_copy(rows0_v.at[pl.ds(0, RPS // 2)],
                    agg_sp.at[pl.ds(sid * RPS + RPS // 2, RPS // 2)])
    plsc.subcore_barrier()

    base0 = sid * ECS

    def _pair(k, _):
        @pl.when(k > 0)
        def _():
            for b in range(2):
                pltpu.make_async_copy(rows[b], agg_sp.at[cidx[b]], ss[b]).wait()
        base = base0 + k * (2 * CH)
        pltpu.sync_copy(r_hbm.at[pl.ds(base, CH)], ridx[0])
        pltpu.sync_copy(c_hbm.at[pl.ds(base, CH)], cidx[0])
        g0 = pltpu.async_copy(hp_my.at[ridx[0]], rows[0], sg[0])
        pltpu.sync_copy(r_hbm.at[pl.ds(base + CH, CH)], ridx[1])
        pltpu.sync_copy(c_hbm.at[pl.ds(base + CH, CH)], cidx[1])
        g1 = pltpu.async_copy(hp_my.at[ridx[1]], rows[1], sg[1])
        g0.wait()
        pltpu.async_copy(rows[0], agg_sp.at[cidx[0]], ss[0], add=True)
        g1.wait()
        pltpu.async_copy(rows[1], agg_sp.at[cidx[1]], ss[1], add=True)
        return 0
    lax.fori_loop(0, NPAIR, _pair, 0)
    for b in range(2):
        pltpu.make_async_copy(rows[b], agg_sp.at[cidx[b]], ss[b]).wait()
    plsc.subcore_barrier()

    row0 = sid * RPS
    pltpu.sync_copy(agg_sp.at[pl.ds(row0, RPS // 2)], rows0_v.at[pl.ds(0, RPS // 2)])
    pltpu.sync_copy(rows0_v.at[pl.ds(0, RPS // 2)], out_hbm.at[cid, pl.ds(row0, RPS // 2)])
    pltpu.sync_copy(agg_sp.at[pl.ds(row0 + RPS // 2, RPS // 2)],
                    rows1_v.at[pl.ds(0, RPS // 2)])
    pltpu.sync_copy(rows1_v.at[pl.ds(0, RPS // 2)],
                    out_hbm.at[cid, pl.ds(row0 + RPS // 2, RPS // 2)])


def _stats_body(x_ref, s1_ref, s2_ref):
    i = pl.program_id(0)

    @pl.when(i == 0)
    def _():
        s1_ref[...] = jnp.zeros_like(s1_ref)
        s2_ref[...] = jnp.zeros_like(s2_ref)
    xb = x_ref[...]
    s1_ref[...] += jnp.sum(xb, axis=0, keepdims=True)
    s2_ref[...] += jnp.sum(xb * xb, axis=0, keepdims=True)


_stats_call = pl.pallas_call(
    _stats_body,
    grid=(GRID,),
    in_specs=[pl.BlockSpec((BLK, F), lambda i: (i, 0))],
    out_specs=[pl.BlockSpec((1, F), lambda i: (0, 0)),
               pl.BlockSpec((1, F), lambda i: (0, 0))],
    out_shape=[jax.ShapeDtypeStruct((1, F), jnp.float32),
               jax.ShapeDtypeStruct((1, F), jnp.float32)],
)


def _bn_coeffs(s1, s2, g, b):
    m = s1 * (1.0 / N)
    v = s2 * (1.0 / N) - m * m
    sc = g * lax.rsqrt(v + EPS)
    return sc, b - m * sc


def _feat_body(x_ref, s1_ref, s2_ref, g_ref, b_ref, w_ref, h_ref, t1_ref, t2_ref):
    i = pl.program_id(0)
    sc, sh = _bn_coeffs(s1_ref[...], s2_ref[...], g_ref[...], b_ref[...])
    t = x_ref[...] * sc + sh
    h = jnp.maximum(jnp.dot(t, w_ref[...], preferred_element_type=jnp.float32), 0.0)
    h_ref[...] = h

    @pl.when(i == 0)
    def _():
        t1_ref[...] = jnp.zeros_like(t1_ref)
        t2_ref[...] = jnp.zeros_like(t2_ref)
    t1_ref[...] += jnp.sum(h, axis=0, keepdims=True)
    t2_ref[...] += jnp.sum(h * h, axis=0, keepdims=True)


_feat_call = pl.pallas_call(
    _feat_body,
    grid=(GRID,),
    in_specs=[pl.BlockSpec((BLK, F), lambda i: (i, 0)),
              pl.BlockSpec((1, F), lambda i: (0, 0)),
              pl.BlockSpec((1, F), lambda i: (0, 0)),
              pl.BlockSpec((1, F), lambda i: (0, 0)),
              pl.BlockSpec((1, F), lambda i: (0, 0)),
              pl.BlockSpec((F, F), lambda i: (0, 0))],
    out_specs=[pl.BlockSpec((BLK, F), lambda i: (i, 0)),
               pl.BlockSpec((1, F), lambda i: (0, 0)),
               pl.BlockSpec((1, F), lambda i: (0, 0))],
    out_shape=[jax.ShapeDtypeStruct((N, F), jnp.float32),
               jax.ShapeDtypeStruct((1, F), jnp.float32),
               jax.ShapeDtypeStruct((1, F), jnp.float32)],
)


def _pre_body(h_ref, s1_ref, s2_ref, g_ref, b_ref, w_ref, d0_ref, d1_ref, hp_ref):
    sc, sh = _bn_coeffs(s1_ref[...], s2_ref[...], g_ref[...], b_ref[...])
    t = h_ref[...] * sc + sh
    u = jnp.dot(t, w_ref[...], preferred_element_type=jnp.float32)
    dis = lax.rsqrt(d0_ref[...] + d1_ref[...] + 1.0)
    hp = u * dis
    hp_ref[0] = hp[:, :FH]
    hp_ref[1] = hp[:, FH:]


_pre_call = pl.pallas_call(
    _pre_body,
    grid=(GRID,),
    in_specs=[pl.BlockSpec((BLK, F), lambda i: (i, 0)),
              pl.BlockSpec((1, F), lambda i: (0, 0)),
              pl.BlockSpec((1, F), lambda i: (0, 0)),
              pl.BlockSpec((1, F), lambda i: (0, 0)),
              pl.BlockSpec((1, F), lambda i: (0, 0)),
              pl.BlockSpec((F, F), lambda i: (0, 0)),
              pl.BlockSpec((BLK, 1), lambda i: (i, 0)),
              pl.BlockSpec((BLK, 1), lambda i: (i, 0))],
    out_specs=pl.BlockSpec((2, BLK, FH), lambda i: (0, i, 0)),
    out_shape=jax.ShapeDtypeStruct((2, N, FH), jnp.float32),
)


def _post_body(a_ref, hp_ref, d0_ref, d1_ref, b_ref, h_ref, t1_ref, t2_ref):
    i = pl.program_id(0)
    dis = lax.rsqrt(d0_ref[...] + d1_ref[...] + 1.0)
    agg = jnp.concatenate([a_ref[0], a_ref[1]], axis=1)
    hp = jnp.concatenate([hp_ref[0], hp_ref[1]], axis=1)
    h = jnp.maximum((agg + hp) * dis + b_ref[...], 0.0)
    h_ref[...] = h

    @pl.when(i == 0)
    def _():
        t1_ref[...] = jnp.zeros_like(t1_ref)
        t2_ref[...] = jnp.zeros_like(t2_ref)
    t1_ref[...] += jnp.sum(h, axis=0, keepdims=True)
    t2_ref[...] += jnp.sum(h * h, axis=0, keepdims=True)


_post_call = pl.pallas_call(
    _post_body,
    grid=(GRID,),
    in_specs=[pl.BlockSpec((2, BLK, FH), lambda i: (0, i, 0)),
              pl.BlockSpec((2, BLK, FH), lambda i: (0, i, 0)),
              pl.BlockSpec((BLK, 1), lambda i: (i, 0)),
              pl.BlockSpec((BLK, 1), lambda i: (i, 0)),
              pl.BlockSpec((1, F), lambda i: (0, 0))],
    out_specs=[pl.BlockSpec((BLK, F), lambda i: (i, 0)),
               pl.BlockSpec((1, F), lambda i: (0, 0)),
               pl.BlockSpec((1, F), lambda i: (0, 0))],
    out_shape=[jax.ShapeDtypeStruct((N, F), jnp.float32),
               jax.ShapeDtypeStruct((1, F), jnp.float32),
               jax.ShapeDtypeStruct((1, F), jnp.float32)],
)


def _pool_body(h_ref, bat_ref, p_ref):
    i = pl.program_id(0)
    bat = bat_ref[0]
    gid = lax.broadcasted_iota(jnp.int32, (NG, BLK), 0)
    mask = jnp.where(bat == gid, 1.0, 0.0)

    @pl.when(i == 0)
    def _():
        p_ref[...] = jnp.zeros_like(p_ref)
    p_ref[...] += jnp.dot(mask, h_ref[...], preferred_element_type=jnp.float32)


_pool_call = pl.pallas_call(
    _pool_body,
    grid=(GRID,),
    in_specs=[pl.BlockSpec((BLK, F), lambda i: (i, 0)),
              pl.BlockSpec((1, 1, BLK), lambda i: (i, 0, 0))],
    out_specs=pl.BlockSpec((NG, F), lambda i: (0, 0)),
    out_shape=jax.ShapeDtypeStruct((NG, F), jnp.float32),
)


def _head_body(p_ref, g1_ref, b1_ref, wf_ref, bf_ref, g2_ref, b2_ref,
               wc_ref, bc_ref, o_ref):
    p = p_ref[...]
    m = jnp.mean(p, axis=0, keepdims=True)
    v = jnp.mean((p - m) * (p - m), axis=0, keepdims=True)
    h = g1_ref[...] * (p - m) * lax.rsqrt(v + EPS) + b1_ref[...]
    h = jnp.maximum(jnp.dot(h, wf_ref[...], preferred_element_type=jnp.float32)
                    + bf_ref[...], 0.0)
    m2 = jnp.mean(h, axis=0, keepdims=True)
    v2 = jnp.mean((h - m2) * (h - m2), axis=0, keepdims=True)
    h = g2_ref[...] * (h - m2) * lax.rsqrt(v2 + EPS) + b2_ref[...]
    lg = jnp.dot(h, wc_ref[...], preferred_element_type=jnp.float32) + bc_ref[...]
    mx = jnp.max(lg, axis=-1, keepdims=True)
    lse = jnp.log(jnp.sum(jnp.exp(lg - mx), axis=-1, keepdims=True)) + mx
    o_ref[...] = lg - lse


_head_call = pl.pallas_call(
    _head_body,
    out_shape=jax.ShapeDtypeStruct((NG, NCLS), jnp.float32),
)


def kernel(x, edge_index, batch, bn_feat_g, bn_feat_b, W_feat,
           bnc_g0, bnc_b0, Wc0, bc0, bnc_g1, bnc_b1, Wc1, bc1,
           bnc_g2, bnc_b2, Wc2, bc2, bn_fc_g, bn_fc_b, W_fc, b_fc,
           bn_hid_g, bn_hid_b, W_cls, b_cls):
    r = edge_index[0]
    c = edge_index[1]

    degp = _deg_call(r)
    d0 = degp[0].reshape(NPAD, 1)[:N]
    d1 = degp[1].reshape(NPAD, 1)[:N]

    s1, s2 = _stats_call(x)
    h, t1, t2 = _feat_call(x, s1, s2, bn_feat_g.reshape(1, F),
                           bn_feat_b.reshape(1, F), W_feat)

    gstk = jnp.stack([bnc_g0, bnc_g1, bnc_g2]).reshape(3, 1, F)
    bstk = jnp.stack([bnc_b0, bnc_b1, bnc_b2]).reshape(3, 1, F)
    Wstk = jnp.stack([Wc0, Wc1, Wc2])
    bbstk = jnp.stack([bc0, bc1, bc2]).reshape(3, 1, F)

    def _layer(carry, xs):
        hc, t1c, t2c = carry
        g, b, W, bb = xs
        hp = _pre_call(hc, t1c, t2c, g, b, W, d0, d1)
        aggp = _agg_call(hp, r, c)
        return tuple(_post_call(aggp[:, :N], hp, d0, d1, bb)), None

    (h, t1, t2), _ = lax.scan(_layer, (h, t1, t2), (gstk, bstk, Wstk, bbstk))

    bat3 = batch.reshape(GRID, 1, BLK)
    pooled = _pool_call(h, bat3)

    return _head_call(pooled, bn_fc_g.reshape(1, F), bn_fc_b.reshape(1, F),
                      W_fc, b_fc.reshape(1, F), bn_hid_g.reshape(1, F),
                      bn_hid_b.reshape(1, F), W_cls, b_cls.reshape(1, NCLS))
